# baseline (speedup 1.0000x reference)
import functools

import jax
import jax.numpy as jnp
from jax import lax
from jax.experimental import pallas as pl
from jax.experimental.pallas import tpu as pltpu
from jax.experimental.pallas import tpu_sc as plsc

_DN = (((1,), (1,)), ((), ()))


def _ew_body(x_ref, gw_ref, nw_ref, n_ref, ew_ref):
    xb = x_ref[...]
    clean = jax.lax.dot_general(xb, gw_ref[...], _DN,
                                preferred_element_type=jnp.float32)
    raw = jax.lax.dot_general(xb, nw_ref[...], _DN,
                              preferred_element_type=jnp.float32)
    ew_ref[...] = clean + n_ref[...] * jax.nn.softplus(raw)


def _expert_weights(x, gate_weights, noise_weights, noise):
    n_tokens, d_model = x.shape
    n_experts = gate_weights.shape[0]
    bt = 4096
    return pl.pallas_call(
        _ew_body,
        grid=(n_tokens // bt,),
        in_specs=[
            pl.BlockSpec((bt, d_model), lambda i: (i, 0)),
            pl.BlockSpec((n_experts, d_model), lambda i: (0, 0)),
            pl.BlockSpec((n_experts, d_model), lambda i: (0, 0)),
            pl.BlockSpec((bt, n_experts), lambda i: (i, 0)),
        ],
        out_specs=pl.BlockSpec((bt, n_experts), lambda i: (i, 0)),
        out_shape=jax.ShapeDtypeStruct((n_tokens, n_experts), jnp.float32),
    )(x, gate_weights, noise_weights, noise)


def _sc_route(ew):
    n, e = ew.shape
    info = plsc.get_sparse_core_info()
    n_cores = info.num_cores
    nw = n_cores * info.num_subcores
    rpw = n // nw
    chunk = 512
    mesh = plsc.VectorSubcoreMesh(core_axis_name="c", subcore_axis_name="s")

    @functools.partial(
        pl.kernel,
        out_type=jax.ShapeDtypeStruct((n * e,), jnp.float32),
        mesh=mesh,
        compiler_params=pltpu.CompilerParams(needs_layout_passes=False),
        scratch_types=[
            pltpu.VMEM((chunk * e,), jnp.float32),
            pltpu.VMEM((chunk * e,), jnp.float32),
        ],
    )
    def route(ew_hbm, out_hbm, bin_, bout):
        wid = lax.axis_index("s") * n_cores + lax.axis_index("c")
        base = wid * rpw

        def do_chunk(ci, carry):
            elt0 = (base + ci * chunk) * e
            pltpu.sync_copy(ew_hbm.at[pl.ds(elt0, chunk * e)], bin_)

            def do_row(r, carry2):
                r0 = r * e
                neg = jnp.full((16,), -jnp.inf, jnp.float32)
                v = [bin_[pl.ds(r0 + 16 * q, 16)] for q in range(4)]
                m1v = jnp.maximum(jnp.maximum(v[0], v[1]),
                                  jnp.maximum(v[2], v[3]))
                m1b = jnp.full((16,), jnp.max(m1v), jnp.float32)
                eq1 = [vq == m1b for vq in v]
                v2 = [jnp.where(eq1[q], neg, v[q]) for q in range(4)]
                m2v = jnp.maximum(jnp.maximum(v2[0], v2[1]),
                                  jnp.maximum(v2[2], v2[3]))
                m2b = jnp.full((16,), jnp.max(m2v), jnp.float32)
                e2 = jnp.exp(m2b - m1b)
                inv = 1.0 / (1.0 + e2)
                p2 = e2 * inv
                zero = jnp.zeros((16,), jnp.float32)
                for q in range(4):
                    eq2 = v2[q] == m2b
                    bout[pl.ds(r0 + 16 * q, 16)] = jnp.where(
                        eq1[q], inv, jnp.where(eq2, p2, zero))
                return carry2

            lax.fori_loop(0, chunk, do_row, 0)
            pltpu.sync_copy(bout, out_hbm.at[pl.ds(elt0, chunk * e)])
            return carry

        lax.fori_loop(0, rpw // chunk, do_chunk, 0)

    return route(ew.reshape(-1)).reshape(n, e)


def kernel(x, gate_weights, noise_weights, noise):
    ew = _expert_weights(x, gate_weights, noise_weights, noise)
    return _sc_route(ew)
